# baseline (speedup 1.0000x reference)
import functools

import jax
import jax.numpy as jnp
from jax import lax
from jax.experimental import pallas as pl
from jax.experimental.pallas import tpu as pltpu
from jax.experimental.pallas import tpu_sc as plsc

_N_NODES = 10000
_N_EDGES = 320000
_D = 128

_NC = 2
_NS = 16
_NW = _NC * _NS
_CHUNK = 128
_CHUNKS_PER_WORKER = 80
_EDGES_PER_WORKER = _CHUNKS_PER_WORKER * _CHUNK
_E_PAD = _EDGES_PER_WORKER * _NW
_ACC_ROWS = 10112
_ROWS_PER_TILE = _ACC_ROWS // _NS
_HALF = _CHUNKS_PER_WORKER // 2


def _sc_aggregate(x, src_p, dst_p):
    mesh = plsc.VectorSubcoreMesh(core_axis_name="c", subcore_axis_name="s")
    npw = _CHUNKS_PER_WORKER

    @functools.partial(
        pl.kernel,
        mesh=mesh,
        out_type=jax.ShapeDtypeStruct((_NC, _ACC_ROWS, _D), jnp.float32),
        scratch_types=[
            pltpu.VMEM_SHARED((_ACC_ROWS, _D), jnp.float32),
            pltpu.VMEM((_HALF, _CHUNK), jnp.int32),
            pltpu.VMEM((_HALF, _CHUNK), jnp.int32),
            pltpu.VMEM((_CHUNK, _D), jnp.float32),
            pltpu.VMEM((_CHUNK, _D), jnp.float32),
            pltpu.SemaphoreType.DMA,
            pltpu.SemaphoreType.DMA,
        ],
    )
    def sc_agg(x_hbm, src_hbm, dst_hbm, out_hbm, acc,
               sidx, didx, rows0, rows1, sem0, sem1):
        c = lax.axis_index("c")
        s = lax.axis_index("s")
        w = c * _NS + s

        zero = jnp.zeros((16,), jnp.float32)

        def zrow(i, carry):
            for t in range(_D // 16):
                rows0[i, pl.ds(t * 16, 16)] = zero
            return carry

        lax.fori_loop(0, _CHUNK, zrow, 0)

        for t in range(_ROWS_PER_TILE // _CHUNK):
            pltpu.sync_copy(
                rows0, acc.at[pl.ds(s * _ROWS_PER_TILE + t * _CHUNK, _CHUNK)]
            )
        rem = _ROWS_PER_TILE % _CHUNK
        pltpu.sync_copy(
            rows0.at[pl.ds(0, rem)],
            acc.at[pl.ds(s * _ROWS_PER_TILE + _ROWS_PER_TILE - rem, rem)],
        )
        plsc.subcore_barrier()

        for h in range(2):
            hbase = w * npw + h * _HALF
            pltpu.sync_copy(src_hbm.at[pl.ds(hbase, _HALF)], sidx)
            pltpu.sync_copy(dst_hbm.at[pl.ds(hbase, _HALF)], didx)

            def step(j, carry):
                pltpu.async_copy(x_hbm.at[sidx.at[j]], rows0, sem0).wait()
                pltpu.sync_copy(rows0, acc.at[didx.at[j]], add=True)
                return carry

            lax.fori_loop(0, _HALF, step, 0)

        plsc.subcore_barrier()

        pltpu.sync_copy(
            acc.at[pl.ds(s * _ROWS_PER_TILE, _ROWS_PER_TILE)],
            out_hbm.at[c].at[pl.ds(s * _ROWS_PER_TILE, _ROWS_PER_TILE)],
        )

    return sc_agg(x, src_p, dst_p)


_BLK = 2000


def _tc_body(p_ref, w_ref, o_ref):
    s = p_ref[0] + p_ref[1]
    o_ref[...] = jnp.dot(s, w_ref[...], preferred_element_type=jnp.float32)


def _tc_combine(partials, w):
    return pl.pallas_call(
        _tc_body,
        grid=(_N_NODES // _BLK,),
        in_specs=[
            pl.BlockSpec((_NC, _BLK, _D), lambda i: (0, i, 0)),
            pl.BlockSpec((_D, _D), lambda i: (0, 0)),
        ],
        out_specs=pl.BlockSpec((_BLK, _D), lambda i: (i, 0)),
        out_shape=jax.ShapeDtypeStruct((_N_NODES, _D), jnp.float32),
    )(partials, w)


def kernel(x, edge_index, weight_low):
    src = edge_index[0]
    dst = edge_index[1]
    pad = _E_PAD - _N_EDGES
    shp = (_NW * _CHUNKS_PER_WORKER, _CHUNK)
    src_p = jnp.concatenate([src, jnp.zeros((pad,), jnp.int32)]).reshape(shp)
    dst_p = jnp.concatenate([dst, jnp.full((pad,), _N_NODES, jnp.int32)]).reshape(shp)
    partials = _sc_aggregate(x, src_p, dst_p)
    return _tc_combine(partials, weight_low)


# speedup vs baseline: 1.1136x; 1.1136x over previous
import functools

import jax
import jax.numpy as jnp
from jax import lax
from jax.experimental import pallas as pl
from jax.experimental.pallas import tpu as pltpu
from jax.experimental.pallas import tpu_sc as plsc

_N_NODES = 10000
_N_EDGES = 320000
_D = 128

_NC = 2
_NS = 16
_NW = _NC * _NS
_CHUNK = 128
_CHUNKS_PER_WORKER = 80
_EDGES_PER_WORKER = _CHUNKS_PER_WORKER * _CHUNK
_E_PAD = _EDGES_PER_WORKER * _NW
_ACC_ROWS = 10112
_ROWS_PER_TILE = _ACC_ROWS // _NS


def _sc_aggregate(x, src_p, dst_p):
    mesh = plsc.VectorSubcoreMesh(core_axis_name="c", subcore_axis_name="s")
    npw = _CHUNKS_PER_WORKER

    @functools.partial(
        pl.kernel,
        mesh=mesh,
        out_type=jax.ShapeDtypeStruct((_NC, _ACC_ROWS, _D), jnp.float32),
        scratch_types=[
            pltpu.VMEM_SHARED((_ACC_ROWS, _D), jnp.float32),
            pltpu.VMEM((_CHUNK,), jnp.int32),
            pltpu.VMEM((_CHUNK,), jnp.int32),
            pltpu.VMEM((_CHUNK,), jnp.int32),
            pltpu.VMEM((_CHUNK,), jnp.int32),
            pltpu.VMEM((_CHUNK, _D), jnp.float32),
            pltpu.VMEM((_CHUNK, _D), jnp.float32),
            pltpu.SemaphoreType.DMA,
            pltpu.SemaphoreType.DMA,
            pltpu.SemaphoreType.DMA,
            pltpu.SemaphoreType.DMA,
        ],
    )
    def sc_agg(x_hbm, src_hbm, dst_hbm, out_hbm, acc,
               sidx0, didx0, sidx1, didx1, rows0, rows1,
               semg0, semg1, semi0, semi1):
        sidx = [sidx0, sidx1]
        didx = [didx0, didx1]
        rows = [rows0, rows1]
        semg = [semg0, semg1]
        semi = [semi0, semi1]
        c = lax.axis_index("c")
        s = lax.axis_index("s")
        w = c * _NS + s

        zero = jnp.zeros((16,), jnp.float32)

        def zrow(i, carry):
            for t in range(_D // 16):
                rows0[i, pl.ds(t * 16, 16)] = zero
            return carry

        lax.fori_loop(0, _CHUNK, zrow, 0)

        for t in range(_ROWS_PER_TILE // _CHUNK):
            pltpu.sync_copy(
                rows0, acc.at[pl.ds(s * _ROWS_PER_TILE + t * _CHUNK, _CHUNK)]
            )
        rem = _ROWS_PER_TILE % _CHUNK
        pltpu.sync_copy(
            rows0.at[pl.ds(0, rem)],
            acc.at[pl.ds(s * _ROWS_PER_TILE + _ROWS_PER_TILE - rem, rem)],
        )
        plsc.subcore_barrier()

        base = w * _EDGES_PER_WORKER

        def load_idx(j, b):
            off = base + j * _CHUNK
            pltpu.async_copy(src_hbm.at[pl.ds(off, _CHUNK)], sidx[b], semi[b])
            pltpu.async_copy(dst_hbm.at[pl.ds(off, _CHUNK)], didx[b], semi[b])

        def wait_idx(j, b):
            off = base + j * _CHUNK
            pltpu.make_async_copy(
                src_hbm.at[pl.ds(off, _CHUNK)], sidx[b], semi[b]).wait()
            pltpu.make_async_copy(
                dst_hbm.at[pl.ds(off, _CHUNK)], didx[b], semi[b]).wait()

        load_idx(0, 0)
        wait_idx(0, 0)
        pltpu.async_copy(x_hbm.at[sidx[0]], rows0, semg[0])
        load_idx(1, 1)

        def halfstep(j, b):
            nb = 1 - b
            wait_idx(j + 1, nb)
            pltpu.async_copy(x_hbm.at[sidx[nb]], rows[nb], semg[nb])
            pltpu.make_async_copy(x_hbm.at[sidx[b]], rows[b], semg[b]).wait()
            pltpu.sync_copy(rows[b], acc.at[didx[b]], add=True)
            load_idx(j + 2, b)

        def step(i, carry):
            halfstep(i * 2, 0)
            halfstep(i * 2 + 1, 1)
            return carry

        lax.fori_loop(0, npw // 2 - 1, step, 0)
        j = npw - 2
        wait_idx(j + 1, 1)
        pltpu.async_copy(x_hbm.at[sidx[1]], rows1, semg[1])
        pltpu.make_async_copy(x_hbm.at[sidx[0]], rows0, semg[0]).wait()
        pltpu.sync_copy(rows0, acc.at[didx[0]], add=True)
        pltpu.make_async_copy(x_hbm.at[sidx[1]], rows1, semg[1]).wait()
        pltpu.sync_copy(rows1, acc.at[didx[1]], add=True)
        plsc.subcore_barrier()

        pltpu.sync_copy(
            acc.at[pl.ds(s * _ROWS_PER_TILE, _ROWS_PER_TILE)],
            out_hbm.at[c].at[pl.ds(s * _ROWS_PER_TILE, _ROWS_PER_TILE)],
        )

    return sc_agg(x, src_p, dst_p)


_BLK = 2000


def _tc_body(p_ref, w_ref, o_ref):
    s = p_ref[0] + p_ref[1]
    o_ref[...] = jnp.dot(s, w_ref[...], preferred_element_type=jnp.float32)


def _tc_combine(partials, w):
    return pl.pallas_call(
        _tc_body,
        grid=(_N_NODES // _BLK,),
        in_specs=[
            pl.BlockSpec((_NC, _BLK, _D), lambda i: (0, i, 0)),
            pl.BlockSpec((_D, _D), lambda i: (0, 0)),
        ],
        out_specs=pl.BlockSpec((_BLK, _D), lambda i: (i, 0)),
        out_shape=jax.ShapeDtypeStruct((_N_NODES, _D), jnp.float32),
    )(partials, w)


def kernel(x, edge_index, weight_low):
    src = edge_index[0]
    dst = edge_index[1]
    pad = _E_PAD - _N_EDGES
    src_p = jnp.concatenate([src, jnp.zeros((pad,), jnp.int32)])
    dst_p = jnp.concatenate([dst, jnp.full((pad,), _N_NODES, jnp.int32)])
    partials = _sc_aggregate(x, src_p, dst_p)
    return _tc_combine(partials, weight_low)


# speedup vs baseline: 1.4220x; 1.2769x over previous
import functools

import jax
import jax.numpy as jnp
from jax import lax
from jax.experimental import pallas as pl
from jax.experimental.pallas import tpu as pltpu
from jax.experimental.pallas import tpu_sc as plsc

_N_NODES = 10000
_N_EDGES = 320000
_D = 128

_NC = 2
_NS = 16
_NW = _NC * _NS
_CHUNK = 128
_CHUNKS_PER_WORKER = 79
_EDGES_PER_WORKER = _CHUNKS_PER_WORKER * _CHUNK
_E_PAD = _EDGES_PER_WORKER * _NW
_ACC_ROWS = 10112
_ROWS_PER_TILE = _ACC_ROWS // _NS


def _sc_aggregate(x, eidx_p):
    mesh = plsc.VectorSubcoreMesh(core_axis_name="c", subcore_axis_name="s")
    npw = _CHUNKS_PER_WORKER

    @functools.partial(
        pl.kernel,
        mesh=mesh,
        out_type=jax.ShapeDtypeStruct((_NC, _ACC_ROWS, _D), jnp.float32),
        scratch_types=[
            pltpu.VMEM_SHARED((_ACC_ROWS, _D), jnp.float32),
            pltpu.VMEM((2, _CHUNK), jnp.int32),
            pltpu.VMEM((_CHUNK, _D), jnp.float32),
            pltpu.SemaphoreType.DMA,
        ],
    )
    def sc_agg(x_hbm, eidx_hbm, out_hbm, acc, eidx, rows, sem):
        c = lax.axis_index("c")
        s = lax.axis_index("s")
        w = c * _NS + s

        zero = jnp.zeros((16,), jnp.float32)

        def zrow(i, carry):
            for t in range(_D // 16):
                rows[i, pl.ds(t * 16, 16)] = zero
            return carry

        lax.fori_loop(0, _CHUNK, zrow, 0)

        for t in range(_ROWS_PER_TILE // _CHUNK):
            pltpu.sync_copy(
                rows, acc.at[pl.ds(s * _ROWS_PER_TILE + t * _CHUNK, _CHUNK)]
            )
        rem = _ROWS_PER_TILE % _CHUNK
        pltpu.sync_copy(
            rows.at[pl.ds(0, rem)],
            acc.at[pl.ds(s * _ROWS_PER_TILE + _ROWS_PER_TILE - rem, rem)],
        )
        plsc.subcore_barrier()

        base = w * npw

        def step(j, carry):
            pltpu.sync_copy(eidx_hbm.at[base + j], eidx)
            pltpu.async_copy(x_hbm.at[eidx.at[0]], rows, sem).wait()
            pltpu.sync_copy(rows, acc.at[eidx.at[1]], add=True)
            return carry

        lax.fori_loop(0, npw, step, 0)
        plsc.subcore_barrier()

        pltpu.sync_copy(
            acc.at[pl.ds(s * _ROWS_PER_TILE, _ROWS_PER_TILE)],
            out_hbm.at[c].at[pl.ds(s * _ROWS_PER_TILE, _ROWS_PER_TILE)],
        )

    return sc_agg(x, eidx_p)


_BLK = 2000


def _tc_body(p_ref, w_ref, o_ref):
    s = p_ref[0] + p_ref[1]
    o_ref[...] = jnp.dot(s, w_ref[...], preferred_element_type=jnp.float32)


def _tc_combine(partials, w):
    return pl.pallas_call(
        _tc_body,
        grid=(_N_NODES // _BLK,),
        in_specs=[
            pl.BlockSpec((_NC, _BLK, _D), lambda i: (0, i, 0)),
            pl.BlockSpec((_D, _D), lambda i: (0, 0)),
        ],
        out_specs=pl.BlockSpec((_BLK, _D), lambda i: (i, 0)),
        out_shape=jax.ShapeDtypeStruct((_N_NODES, _D), jnp.float32),
    )(partials, w)


def kernel(x, edge_index, weight_low):
    src = edge_index[0]
    dst = edge_index[1]
    pad = _E_PAD - _N_EDGES
    src_p = jnp.concatenate([src, jnp.zeros((pad,), jnp.int32)])
    dst_p = jnp.concatenate([dst, jnp.full((pad,), _N_NODES, jnp.int32)])
    nchunks = _NW * _CHUNKS_PER_WORKER
    eidx_p = jnp.stack(
        [src_p.reshape(nchunks, _CHUNK), dst_p.reshape(nchunks, _CHUNK)], axis=1
    )
    partials = _sc_aggregate(x, eidx_p)
    return _tc_combine(partials, weight_low)


# speedup vs baseline: 2.0962x; 1.4742x over previous
import functools

import jax
import jax.numpy as jnp
from jax import lax
from jax.experimental import pallas as pl
from jax.experimental.pallas import tpu as pltpu
from jax.experimental.pallas import tpu_sc as plsc

_N_NODES = 10000
_N_EDGES = 320000
_D = 128
_DH = 64

_NC = 2
_NS = 16
_CHUNK = 128
_QUAD = 4
_CHUNKS_PER_TILE = 160
_E_PAD = _CHUNKS_PER_TILE * _CHUNK * _NS
_ACC_ROWS = 10240
_ROWS_PER_TILE = _ACC_ROWS // _NS


def _sc_aggregate(sup2, eidx_p):
    mesh = plsc.VectorSubcoreMesh(core_axis_name="c", subcore_axis_name="s")
    npt = _CHUNKS_PER_TILE

    @functools.partial(
        pl.kernel,
        mesh=mesh,
        out_type=jax.ShapeDtypeStruct((_NC, _ACC_ROWS, _DH), jnp.float32),
        scratch_types=[
            pltpu.VMEM_SHARED((_ACC_ROWS, _DH), jnp.float32),
            pltpu.VMEM_SHARED((_ACC_ROWS, _DH), jnp.float32),
            pltpu.VMEM((_QUAD, 2, _CHUNK), jnp.int32),
            pltpu.VMEM((_CHUNK, _DH), jnp.float32),
            pltpu.SemaphoreType.DMA,
        ],
    )
    def sc_agg(sup_hbm, eidx_hbm, out_hbm, sup_sp, acc, eidx, rows, sem):
        c = lax.axis_index("c")
        s = lax.axis_index("s")

        zero = jnp.zeros((16,), jnp.float32)

        def zrow(i, carry):
            for t in range(_DH // 16):
                rows[i, pl.ds(t * 16, 16)] = zero
            return carry

        lax.fori_loop(0, _CHUNK, zrow, 0)

        tbase = s * _ROWS_PER_TILE
        pltpu.sync_copy(
            sup_hbm.at[c].at[pl.ds(tbase, _ROWS_PER_TILE)],
            sup_sp.at[pl.ds(tbase, _ROWS_PER_TILE)],
        )
        for t in range(_ROWS_PER_TILE // _CHUNK):
            pltpu.sync_copy(rows, acc.at[pl.ds(tbase + t * _CHUNK, _CHUNK)])
        plsc.subcore_barrier()

        base = s * (npt // _QUAD)

        def step(q, carry):
            pltpu.sync_copy(eidx_hbm.at[base + q], eidx)
            for t in range(_QUAD):
                pltpu.async_copy(sup_sp.at[eidx.at[t].at[0]], rows, sem).wait()
                pltpu.sync_copy(rows, acc.at[eidx.at[t].at[1]], add=True)
            return carry

        lax.fori_loop(0, npt // _QUAD, step, 0)
        plsc.subcore_barrier()

        pltpu.sync_copy(
            acc.at[pl.ds(tbase, _ROWS_PER_TILE)],
            out_hbm.at[c].at[pl.ds(tbase, _ROWS_PER_TILE)],
        )

    return sc_agg(sup2, eidx_p)


_MM_BLK = 2048


def _tc_mm_body(x_ref, w_ref, o_ref):
    sup = jnp.dot(x_ref[...], w_ref[...], preferred_element_type=jnp.float32)
    o_ref[0] = sup[:, :_DH]
    o_ref[1] = sup[:, _DH:]


def _tc_matmul_split(xp, w):
    return pl.pallas_call(
        _tc_mm_body,
        grid=(_ACC_ROWS // _MM_BLK,),
        in_specs=[
            pl.BlockSpec((_MM_BLK, _D), lambda i: (i, 0)),
            pl.BlockSpec((_D, _D), lambda i: (0, 0)),
        ],
        out_specs=pl.BlockSpec((_NC, _MM_BLK, _DH), lambda i: (0, i, 0)),
        out_shape=jax.ShapeDtypeStruct((_NC, _ACC_ROWS, _DH), jnp.float32),
    )(xp, w)


_BLK = 2000


def _tc_cat_body(p_ref, o_ref):
    o_ref[:, :_DH] = p_ref[0]
    o_ref[:, _DH:] = p_ref[1]


def _tc_combine(partials):
    return pl.pallas_call(
        _tc_cat_body,
        grid=(_N_NODES // _BLK,),
        in_specs=[pl.BlockSpec((_NC, _BLK, _DH), lambda i: (0, i, 0))],
        out_specs=pl.BlockSpec((_BLK, _D), lambda i: (i, 0)),
        out_shape=jax.ShapeDtypeStruct((_N_NODES, _D), jnp.float32),
    )(partials)


def kernel(x, edge_index, weight_low):
    src = edge_index[0]
    dst = edge_index[1]
    pad = _E_PAD - _N_EDGES
    src_p = jnp.concatenate([src, jnp.zeros((pad,), jnp.int32)])
    dst_p = jnp.concatenate([dst, jnp.full((pad,), _N_NODES, jnp.int32)])
    nq = _NS * (_CHUNKS_PER_TILE // _QUAD)
    eidx_p = jnp.stack(
        [src_p.reshape(nq, _QUAD, _CHUNK), dst_p.reshape(nq, _QUAD, _CHUNK)],
        axis=2,
    )
    xp = jnp.pad(x, ((0, _ACC_ROWS - _N_NODES), (0, 0)))
    sup2 = _tc_matmul_split(xp, weight_low)
    partials = _sc_aggregate(sup2, eidx_p)
    return _tc_combine(partials)


# speedup vs baseline: 2.0986x; 1.0011x over previous
import functools

import jax
import jax.numpy as jnp
from jax import lax
from jax.experimental import pallas as pl
from jax.experimental.pallas import tpu as pltpu
from jax.experimental.pallas import tpu_sc as plsc

_N_NODES = 10000
_N_EDGES = 320000
_D = 128
_DH = 64

_NC = 2
_NS = 16
_CHUNK = 128
_QUAD = 4
_CHUNKS_PER_TILE = 160
_E_PAD = _CHUNKS_PER_TILE * _CHUNK * _NS
_ACC_ROWS = 10240
_ROWS_PER_TILE = _ACC_ROWS // _NS


def _sc_aggregate(sup2, eidx_p):
    mesh = plsc.VectorSubcoreMesh(core_axis_name="c", subcore_axis_name="s")
    npt = _CHUNKS_PER_TILE

    @functools.partial(
        pl.kernel,
        mesh=mesh,
        out_type=jax.ShapeDtypeStruct((_NC, _ACC_ROWS, _DH), jnp.float32),
        scratch_types=[
            pltpu.VMEM_SHARED((_ACC_ROWS, _DH), jnp.float32),
            pltpu.VMEM_SHARED((_ACC_ROWS, _DH), jnp.float32),
            pltpu.VMEM((2 * _QUAD, _CHUNK), jnp.int32),
            pltpu.VMEM((_CHUNK, _DH), jnp.float32),
            pltpu.SemaphoreType.DMA,
        ],
    )
    def sc_agg(sup_hbm, eidx_hbm, out_hbm, sup_sp, acc, eidx, rows, sem):
        c = lax.axis_index("c")
        s = lax.axis_index("s")

        zero = jnp.zeros((16,), jnp.float32)

        def zrow(i, carry):
            for t in range(_DH // 16):
                rows[i, pl.ds(t * 16, 16)] = zero
            return carry

        lax.fori_loop(0, _CHUNK, zrow, 0)

        tbase = s * _ROWS_PER_TILE
        pltpu.sync_copy(
            sup_hbm.at[c].at[pl.ds(tbase, _ROWS_PER_TILE)],
            sup_sp.at[pl.ds(tbase, _ROWS_PER_TILE)],
        )
        for t in range(_ROWS_PER_TILE // _CHUNK):
            pltpu.sync_copy(rows, acc.at[pl.ds(tbase + t * _CHUNK, _CHUNK)])
        plsc.subcore_barrier()

        base = s * (npt // _QUAD)

        def step(q, carry):
            pltpu.sync_copy(eidx_hbm.at[base + q], eidx)
            for t in range(_QUAD):
                pltpu.async_copy(sup_sp.at[eidx.at[2 * t]], rows, sem).wait()
                pltpu.sync_copy(rows, acc.at[eidx.at[2 * t + 1]], add=True)
            return carry

        lax.fori_loop(0, npt // _QUAD, step, 0)
        plsc.subcore_barrier()

        pltpu.sync_copy(
            acc.at[pl.ds(tbase, _ROWS_PER_TILE)],
            out_hbm.at[c].at[pl.ds(tbase, _ROWS_PER_TILE)],
        )

    return sc_agg(sup2, eidx_p)


_MM_BLK = 2048


def _tc_mm_body(x_ref, w_ref, o_ref):
    sup = jnp.dot(x_ref[...], w_ref[...], preferred_element_type=jnp.float32)
    o_ref[0] = sup[:, :_DH]
    o_ref[1] = sup[:, _DH:]


def _tc_matmul_split(xp, w):
    return pl.pallas_call(
        _tc_mm_body,
        grid=(_ACC_ROWS // _MM_BLK,),
        in_specs=[
            pl.BlockSpec((_MM_BLK, _D), lambda i: (i, 0)),
            pl.BlockSpec((_D, _D), lambda i: (0, 0)),
        ],
        out_specs=pl.BlockSpec((_NC, _MM_BLK, _DH), lambda i: (0, i, 0)),
        out_shape=jax.ShapeDtypeStruct((_NC, _ACC_ROWS, _DH), jnp.float32),
    )(xp, w)


_BLK = 2000


def _tc_cat_body(p_ref, o_ref):
    o_ref[:, :_DH] = p_ref[0]
    o_ref[:, _DH:] = p_ref[1]


def _tc_combine(partials):
    return pl.pallas_call(
        _tc_cat_body,
        grid=(_N_NODES // _BLK,),
        in_specs=[pl.BlockSpec((_NC, _BLK, _DH), lambda i: (0, i, 0))],
        out_specs=pl.BlockSpec((_BLK, _D), lambda i: (i, 0)),
        out_shape=jax.ShapeDtypeStruct((_N_NODES, _D), jnp.float32),
    )(partials)


def kernel(x, edge_index, weight_low):
    src = edge_index[0]
    dst = edge_index[1]
    pad = _E_PAD - _N_EDGES
    src_p = jnp.concatenate([src, jnp.zeros((pad,), jnp.int32)])
    dst_p = jnp.concatenate([dst, jnp.full((pad,), _N_NODES, jnp.int32)])
    nq = _NS * (_CHUNKS_PER_TILE // _QUAD)
    eidx_p = jnp.stack(
        [src_p.reshape(nq, _QUAD, _CHUNK), dst_p.reshape(nq, _QUAD, _CHUNK)],
        axis=2,
    ).reshape(nq, 2 * _QUAD, _CHUNK)
    xp = jnp.pad(x, ((0, _ACC_ROWS - _N_NODES), (0, 0)))
    sup2 = _tc_matmul_split(xp, weight_low)
    partials = _sc_aggregate(sup2, eidx_p)
    return _tc_combine(partials)


# speedup vs baseline: 2.1458x; 1.0225x over previous
import functools

import jax
import jax.numpy as jnp
from jax import lax
from jax.experimental import pallas as pl
from jax.experimental.pallas import tpu as pltpu
from jax.experimental.pallas import tpu_sc as plsc

_N_NODES = 10000
_N_EDGES = 320000
_D = 128
_DH = 64

_NC = 2
_NS = 16
_CHUNK = 128
_QUAD = 8
_CHUNKS_PER_TILE = 160
_E_PAD = _CHUNKS_PER_TILE * _CHUNK * _NS
_ACC_ROWS = 10240
_ROWS_PER_TILE = _ACC_ROWS // _NS


def _sc_aggregate(sup2, eidx_p):
    mesh = plsc.VectorSubcoreMesh(core_axis_name="c", subcore_axis_name="s")
    npt = _CHUNKS_PER_TILE

    @functools.partial(
        pl.kernel,
        mesh=mesh,
        out_type=jax.ShapeDtypeStruct((_NC, _ACC_ROWS, _DH), jnp.float32),
        scratch_types=[
            pltpu.VMEM_SHARED((_ACC_ROWS, _DH), jnp.float32),
            pltpu.VMEM_SHARED((_ACC_ROWS, _DH), jnp.float32),
            pltpu.VMEM((2 * _QUAD, _CHUNK), jnp.int32),
            pltpu.VMEM((_CHUNK, _DH), jnp.float32),
            pltpu.SemaphoreType.DMA,
        ],
    )
    def sc_agg(sup_hbm, eidx_hbm, out_hbm, sup_sp, acc, eidx, rows, sem):
        c = lax.axis_index("c")
        s = lax.axis_index("s")

        zero = jnp.zeros((16,), jnp.float32)

        def zrow(i, carry):
            for t in range(_DH // 16):
                rows[i, pl.ds(t * 16, 16)] = zero
            return carry

        lax.fori_loop(0, _CHUNK, zrow, 0)

        tbase = s * _ROWS_PER_TILE
        pltpu.sync_copy(
            sup_hbm.at[c].at[pl.ds(tbase, _ROWS_PER_TILE)],
            sup_sp.at[pl.ds(tbase, _ROWS_PER_TILE)],
        )
        for t in range(_ROWS_PER_TILE // _CHUNK):
            pltpu.sync_copy(rows, acc.at[pl.ds(tbase + t * _CHUNK, _CHUNK)])
        plsc.subcore_barrier()

        base = s * (npt // _QUAD)

        def step(q, carry):
            pltpu.sync_copy(eidx_hbm.at[base + q], eidx)
            for t in range(_QUAD):
                pltpu.async_copy(sup_sp.at[eidx.at[2 * t]], rows, sem).wait()
                pltpu.sync_copy(rows, acc.at[eidx.at[2 * t + 1]], add=True)
            return carry

        lax.fori_loop(0, npt // _QUAD, step, 0)
        plsc.subcore_barrier()

        pltpu.sync_copy(
            acc.at[pl.ds(tbase, _ROWS_PER_TILE)],
            out_hbm.at[c].at[pl.ds(tbase, _ROWS_PER_TILE)],
        )

    return sc_agg(sup2, eidx_p)


_MM_BLK = 2048


def _tc_mm_body(x_ref, w_ref, o_ref):
    sup = jnp.dot(x_ref[...], w_ref[...], preferred_element_type=jnp.float32)
    o_ref[0] = sup[:, :_DH]
    o_ref[1] = sup[:, _DH:]


def _tc_matmul_split(xp, w):
    return pl.pallas_call(
        _tc_mm_body,
        grid=(_ACC_ROWS // _MM_BLK,),
        in_specs=[
            pl.BlockSpec((_MM_BLK, _D), lambda i: (i, 0)),
            pl.BlockSpec((_D, _D), lambda i: (0, 0)),
        ],
        out_specs=pl.BlockSpec((_NC, _MM_BLK, _DH), lambda i: (0, i, 0)),
        out_shape=jax.ShapeDtypeStruct((_NC, _ACC_ROWS, _DH), jnp.float32),
    )(xp, w)


_BLK = 2000


def _tc_cat_body(p_ref, o_ref):
    o_ref[:, :_DH] = p_ref[0]
    o_ref[:, _DH:] = p_ref[1]


def _tc_combine(partials):
    return pl.pallas_call(
        _tc_cat_body,
        grid=(_N_NODES // _BLK,),
        in_specs=[pl.BlockSpec((_NC, _BLK, _DH), lambda i: (0, i, 0))],
        out_specs=pl.BlockSpec((_BLK, _D), lambda i: (i, 0)),
        out_shape=jax.ShapeDtypeStruct((_N_NODES, _D), jnp.float32),
    )(partials)


def kernel(x, edge_index, weight_low):
    src = edge_index[0]
    dst = edge_index[1]
    pad = _E_PAD - _N_EDGES
    src_p = jnp.concatenate([src, jnp.zeros((pad,), jnp.int32)])
    dst_p = jnp.concatenate([dst, jnp.full((pad,), _N_NODES, jnp.int32)])
    nq = _NS * (_CHUNKS_PER_TILE // _QUAD)
    eidx_p = jnp.stack(
        [src_p.reshape(nq, _QUAD, _CHUNK), dst_p.reshape(nq, _QUAD, _CHUNK)],
        axis=2,
    ).reshape(nq, 2 * _QUAD, _CHUNK)
    xp = jnp.pad(x, ((0, _ACC_ROWS - _N_NODES), (0, 0)))
    sup2 = _tc_matmul_split(xp, weight_low)
    partials = _sc_aggregate(sup2, eidx_p)
    return _tc_combine(partials)
